# baseline (speedup 1.0000x reference)
import functools

import jax
import jax.numpy as jnp
from jax import lax
from jax.experimental import pallas as pl
from jax.experimental.pallas import tpu as pltpu
from jax.experimental.pallas import tpu_sc as plsc

N = 10000
E = 320000
H = 128
ED = 16
L = 2

NC = 2
NS = 16
NW = NC * NS
EPW = E // NW
CH = 80
NCH = EPW // CH
NPAD = 10240
NPS = NPAD // NS

F32 = jnp.float32
I32 = jnp.int32


def _sc_mesh():
    return plsc.VectorSubcoreMesh(
        core_axis_name="c", subcore_axis_name="s",
        num_cores=NC, num_subcores=NS)


NB = 3
GHEAD = 5


def _gather_pass(a, b, src, dst, px, py, pz, wd2):
    @functools.partial(
        pl.kernel,
        out_type=jax.ShapeDtypeStruct((E, H), F32),
        mesh=_sc_mesh(),
        scratch_types=[
            pltpu.VMEM((N,), F32),
            pltpu.VMEM((N,), F32),
            pltpu.VMEM((N,), F32),
            pltpu.VMEM((H,), F32),
            [pltpu.VMEM((CH, H), F32)] * 2,
            [pltpu.VMEM((CH, H), F32)] * 2,
            [pltpu.VMEM((CH, H), F32)] * 2,
            [pltpu.VMEM((CH,), I32)] * 4,
            [pltpu.VMEM((CH,), I32)] * 4,
            [pltpu.SemaphoreType.DMA] * 2,
            [pltpu.SemaphoreType.DMA] * 2,
            [pltpu.SemaphoreType.DMA] * 2,
            [pltpu.SemaphoreType.DMA] * 4,
            [pltpu.SemaphoreType.DMA] * 4,
        ],
        compiler_params=pltpu.CompilerParams(needs_layout_passes=False),
    )
    def k(a_h, b_h, src_h, dst_h, px_h, py_h, pz_h, w_h, out_h,
          px_v, py_v, pz_v, w_v, av, bv, gv, ivs, ivd,
          sga, sgb, so, sis, sid_):
        cid = lax.axis_index("c")
        sid = lax.axis_index("s")
        base = (cid * NS + sid) * EPW
        pltpu.sync_copy(px_h, px_v)
        pltpu.sync_copy(py_h, py_v)
        pltpu.sync_copy(pz_h, pz_v)
        pltpu.sync_copy(w_h, w_v)
        wrow = [w_v[pl.ds(j * 16, 16)] for j in range(H // 16)]

        def ld_is(c, bi):
            return pltpu.make_async_copy(
                src_h.at[pl.ds(base + c * CH, CH)], ivs[bi], sis[bi])

        def ld_id(c, bi):
            return pltpu.make_async_copy(
                dst_h.at[pl.ds(base + c * CH, CH)], ivd[bi], sid_[bi])

        def gat_a(bf, bi):
            return pltpu.make_async_copy(a_h.at[ivs[bi]], av[bf], sga[bf])

        def gat_b(bf, bi):
            return pltpu.make_async_copy(b_h.at[ivd[bi]], bv[bf], sgb[bf])

        def owr(c, bf):
            return pltpu.make_async_copy(
                gv[bf], out_h.at[pl.ds(base + c * CH, CH)], so[bf])

        def compute(bf, bi):
            def grp(g, carry):
                r0 = g * 16
                si = ivs[bi][pl.ds(r0, 16)]
                di = ivd[bi][pl.ds(r0, 16)]
                rx = plsc.load_gather(px_v, [si]) - plsc.load_gather(px_v, [di])
                ry = plsc.load_gather(py_v, [si]) - plsc.load_gather(py_v, [di])
                rz = plsc.load_gather(pz_v, [si]) - plsc.load_gather(pz_v, [di])
                d2g = rx * rx + ry * ry + rz * rz
                for i in range(16):
                    r = r0 + i
                    d2s = d2g[i]
                    for j in range(H // 16):
                        sl = pl.ds(j * 16, 16)
                        gv[bf][r, sl] = (av[bf][r, sl] + bv[bf][r, sl]
                                         + d2s * wrow[j])
                return carry

            lax.fori_loop(0, CH // 16, grp, 0)

        def step(c, bf, bi):
            bf1, bi1, bi2 = 1 - bf, (bi + 1) % 4, (bi + 2) % 4
            gat_a(bf, bi).wait()
            gat_b(bf, bi).wait()

            def ahead1():
                ld_is(c + 1, bi1).wait()
                ld_id(c + 1, bi1).wait()
                gat_a(bf1, bi1).start()
                gat_b(bf1, bi1).start()

            def ahead2():
                ld_is(c + 2, bi2).start()
                ld_id(c + 2, bi2).start()

            if isinstance(c, int):
                if c + 1 < NCH:
                    ahead1()
                if c + 2 < NCH:
                    ahead2()
                if c >= 2:
                    owr(c - 2, bf).wait()
            else:
                pl.when(c + 1 < NCH)(ahead1)
                pl.when(c + 2 < NCH)(ahead2)
                owr(c - 2, bf).wait()

            compute(bf, bi)
            owr(c, bf).start()

        ld_is(0, 0).start()
        ld_id(0, 0).start()
        ld_is(1, 1).start()
        ld_id(1, 1).start()
        ld_is(0, 0).wait()
        ld_id(0, 0).wait()
        gat_a(0, 0).start()
        gat_b(0, 0).start()
        for c in range(GHEAD):
            step(c, c % 2, c % 4)

        def body(q, carry):
            c0 = GHEAD + 4 * q
            for r in range(4):
                cr = GHEAD + r
                step(c0 + r, cr % 2, cr % 4)
            return carry

        lax.fori_loop(0, (NCH - GHEAD) // 4, body, 0)
        owr(NCH - 2, (NCH - 2) % 2).wait()
        owr(NCH - 1, (NCH - 1) % 2).wait()

    return k(a, b, src, dst, px, py, pz, wd2)


def _scatter_pass(m2, dst, zeros_n):
    @functools.partial(
        pl.kernel,
        out_type=jax.ShapeDtypeStruct((NC, NPAD, H), F32),
        mesh=_sc_mesh(),
        scratch_types=[
            [pltpu.VMEM((CH, H), F32)] * NB,
            [pltpu.VMEM((CH,), I32)] * NB,
            pltpu.VMEM_SHARED((NPAD, H), F32),
            [pltpu.SemaphoreType.DMA] * NB,
            [pltpu.SemaphoreType.DMA] * NB,
            [pltpu.SemaphoreType.DMA] * NB,
        ],
    )
    def k(m2_h, dst_h, zz_h, out_h, mv, iv, agg_sh, sm, si, ss):
        cid = lax.axis_index("c")
        sid = lax.axis_index("s")
        base = (cid * NS + sid) * EPW
        pltpu.sync_copy(zz_h, agg_sh.at[pl.ds(sid * NPS, NPS)])
        plsc.subcore_barrier()

        def ld_m(c, bf):
            return pltpu.make_async_copy(
                m2_h.at[pl.ds(base + c * CH, CH)], mv[bf], sm[bf])

        def ld_i(c, bf):
            return pltpu.make_async_copy(
                dst_h.at[pl.ds(base + c * CH, CH)], iv[bf], si[bf])

        def scat(bf):
            return pltpu.make_async_copy(mv[bf], agg_sh.at[iv[bf]], ss[bf])

        def issue(c, bf):
            ld_m(c, bf).start()
            ld_i(c, bf).start()

        def step(c, bf):
            ld_m(c, bf).wait()
            ld_i(c, bf).wait()
            nb = (bf + NB - 1) % NB

            if isinstance(c, int):
                if c + NB - 1 < NCH:
                    if c >= 1:
                        scat(nb).wait()
                    issue(c + NB - 1, nb)
            else:
                @pl.when(c + NB - 1 < NCH)
                def _():
                    scat(nb).wait()
                    issue(c + NB - 1, nb)

            scat(bf).start(add=True)

        for c in range(NB - 1):
            issue(c, c)
        for c in range(NB - 1):
            step(c, c)

        def body(q, carry):
            c0 = (NB - 1) + NB * q
            for r in range(NB):
                step(c0 + r, (NB - 1 + r) % NB)
            return carry

        lax.fori_loop(0, (NCH - (NB - 1)) // NB, body, 0)
        for bf in range(NB):
            scat(bf).wait()
        plsc.subcore_barrier()
        pltpu.sync_copy(agg_sh.at[pl.ds(sid * NPS, NPS)],
                        out_h.at[cid, pl.ds(sid * NPS, NPS)])

    return k(m2, dst, zeros_n)


BRN = 2000
BRE = 2000


DE = 2 * H + 1 + ED


def _tc_prep0(x, we1):
    def body(x_ref, w_ref, a_ref, b_ref):
        xv = x_ref[...]
        w = w_ref[0]
        a_ref[...] = jnp.dot(xv, w[:H, :], preferred_element_type=F32)
        b_ref[...] = jnp.dot(xv, w[H:2 * H, :], preferred_element_type=F32)

    return pl.pallas_call(
        body,
        grid=(N // BRN,),
        in_specs=[
            pl.BlockSpec((BRN, H), lambda i: (i, 0)),
            pl.BlockSpec((1, DE, H), lambda i: (0, 0, 0)),
        ],
        out_specs=[pl.BlockSpec((BRN, H), lambda i: (i, 0))] * 2,
        out_shape=[jax.ShapeDtypeStruct((N, H), F32)] * 2,
    )(x, we1)


def _tc_edge(l, g, ea, wpack, eprm):
    def body(g_ref, ea_ref, w_ref, prm_ref, o_ref):
        w = w_ref[0]
        prm = prm_ref[0]
        m1 = (g_ref[...]
              + jnp.dot(ea_ref[...], w[:ED, :], preferred_element_type=F32)
              + prm[0:1, :])
        m1 = m1 * lax.logistic(m1)
        m2 = jnp.dot(m1, w[ED:, :], preferred_element_type=F32) + prm[1:2, :]
        o_ref[...] = m2 * lax.logistic(m2)

    return pl.pallas_call(
        body,
        grid=(E // BRE,),
        in_specs=[
            pl.BlockSpec((BRE, H), lambda i: (i, 0)),
            pl.BlockSpec((BRE, ED), lambda i: (i, 0)),
            pl.BlockSpec((1, ED + H, H), lambda i, _l=l: (_l, 0, 0)),
            pl.BlockSpec((1, 2, H), lambda i, _l=l: (_l, 0, 0)),
        ],
        out_specs=pl.BlockSpec((BRE, H), lambda i: (i, 0)),
        out_shape=jax.ShapeDtypeStruct((E, H), F32),
    )(g, ea, wpack, eprm)


def _tc_node(l, h, aggp, wn1, wn2, nprm, we1=None):
    with_prep = we1 is not None

    def body(h_ref, ag_ref, wn1_ref, wn2_ref, prm_ref, *rest):
        wn1 = wn1_ref[0]
        prm = prm_ref[0]
        agg = ag_ref[0] + ag_ref[1]
        t = (jnp.dot(h_ref[...], wn1[:H, :], preferred_element_type=F32)
             + jnp.dot(agg, wn1[H:, :], preferred_element_type=F32)
             + prm[0:1, :])
        t = t * lax.logistic(t)
        hn = jnp.dot(t, wn2_ref[0], preferred_element_type=F32) + prm[1:2, :]
        if with_prep:
            we1_ref, hn_ref, a_ref, b_ref = rest
            w = we1_ref[0]
            hn_ref[...] = hn
            a_ref[...] = jnp.dot(hn, w[:H, :], preferred_element_type=F32)
            b_ref[...] = jnp.dot(hn, w[H:2 * H, :], preferred_element_type=F32)
        else:
            (hn_ref,) = rest
            hn_ref[...] = hn

    blk = lambda i: (i, 0)
    lsel = lambda shape: pl.BlockSpec(shape, lambda i, _l=l: (_l, 0, 0))
    in_specs = [
        pl.BlockSpec((BRN, H), blk),
        pl.BlockSpec((NC, BRN, H), lambda i: (0, i, 0)),
        lsel((1, 2 * H, H)),
        lsel((1, H, H)),
        lsel((1, 2, H)),
    ]
    args = [h, aggp, wn1, wn2, nprm]
    if with_prep:
        in_specs.append(pl.BlockSpec((1, DE, H), lambda i, _l=l: (_l + 1, 0, 0)))
        args.append(we1)
        out_specs = [pl.BlockSpec((BRN, H), blk)] * 3
        out_shape = [jax.ShapeDtypeStruct((N, H), F32)] * 3
    else:
        out_specs = pl.BlockSpec((BRN, H), blk)
        out_shape = jax.ShapeDtypeStruct((N, H), F32)

    return pl.pallas_call(
        body,
        grid=(N // BRN,),
        in_specs=in_specs,
        out_specs=out_specs,
        out_shape=out_shape,
    )(*args)


def kernel(x, pos, edge_index, edge_attr, We1, be1, We2, be2, Wn1, bn1, Wn2, bn2):
    src = edge_index[0]
    dst = edge_index[1]
    px = pos[:, 0]
    py = pos[:, 1]
    pz = pos[:, 2]
    zeros_t = jnp.zeros((NPS, H), F32)
    wpack = jnp.concatenate([We1[:, 2 * H + 1:], We2], axis=1)
    eprm = jnp.stack([be1, be2], axis=1)
    nprm = jnp.stack([bn1, bn2], axis=1)
    wd2 = We1[:, 2 * H]

    a, b = _tc_prep0(x, We1)
    h = x
    for l in range(L):
        g = _gather_pass(a, b, src, dst, px, py, pz, wd2[l])
        m2 = _tc_edge(l, g, edge_attr, wpack, eprm)
        aggp = _scatter_pass(m2, dst, zeros_t)
        if l < L - 1:
            h, a, b = _tc_node(l, h, aggp, Wn1, Wn2, nprm, We1)
        else:
            h = _tc_node(l, h, aggp, Wn1, Wn2, nprm)
    return h, pos


# speedup vs baseline: 1.1192x; 1.1192x over previous
import functools

import jax
import jax.numpy as jnp
from jax import lax
from jax.experimental import pallas as pl
from jax.experimental.pallas import tpu as pltpu
from jax.experimental.pallas import tpu_sc as plsc

N = 10000
E = 320000
H = 128
ED = 16
L = 2

NC = 2
NS = 16
NW = NC * NS
EPW = E // NW
CH = 80
NCH = EPW // CH
NPAD = 10240
NPS = NPAD // NS

F32 = jnp.float32
I32 = jnp.int32


def _sc_mesh():
    return plsc.VectorSubcoreMesh(
        core_axis_name="c", subcore_axis_name="s",
        num_cores=NC, num_subcores=NS)


NB = 3
GHEAD = 5


def _gather_pass(a, b, src, dst, px, py, pz, wd2):
    @functools.partial(
        pl.kernel,
        out_type=jax.ShapeDtypeStruct((E, H), F32),
        mesh=_sc_mesh(),
        scratch_types=[
            pltpu.VMEM((N,), F32),
            pltpu.VMEM((N,), F32),
            pltpu.VMEM((N,), F32),
            pltpu.VMEM((H,), F32),
            [pltpu.VMEM((CH, H), F32)] * 2,
            [pltpu.VMEM((CH, H), F32)] * 2,
            [pltpu.VMEM((CH, H), F32)] * 2,
            [pltpu.VMEM((CH,), I32)] * 4,
            [pltpu.VMEM((CH,), I32)] * 4,
            [pltpu.SemaphoreType.DMA] * 2,
            [pltpu.SemaphoreType.DMA] * 2,
            [pltpu.SemaphoreType.DMA] * 2,
            [pltpu.SemaphoreType.DMA] * 4,
            [pltpu.SemaphoreType.DMA] * 4,
        ],
        compiler_params=pltpu.CompilerParams(needs_layout_passes=False),
    )
    def k(a_h, b_h, src_h, dst_h, px_h, py_h, pz_h, w_h, out_h,
          px_v, py_v, pz_v, w_v, av, bv, gv, ivs, ivd,
          sga, sgb, so, sis, sid_):
        cid = lax.axis_index("c")
        sid = lax.axis_index("s")
        base = (cid * NS + sid) * EPW
        pltpu.sync_copy(px_h, px_v)
        pltpu.sync_copy(py_h, py_v)
        pltpu.sync_copy(pz_h, pz_v)
        pltpu.sync_copy(w_h, w_v)
        wrow = [w_v[pl.ds(j * 16, 16)] for j in range(H // 16)]

        def ld_is(c, bi):
            return pltpu.make_async_copy(
                src_h.at[pl.ds(base + c * CH, CH)], ivs[bi], sis[bi])

        def ld_id(c, bi):
            return pltpu.make_async_copy(
                dst_h.at[pl.ds(base + c * CH, CH)], ivd[bi], sid_[bi])

        def gat_a(bf, bi):
            return pltpu.make_async_copy(a_h.at[ivs[bi]], av[bf], sga[bf])

        def gat_b(bf, bi):
            return pltpu.make_async_copy(b_h.at[ivd[bi]], bv[bf], sgb[bf])

        def owr(c, bf):
            return pltpu.make_async_copy(
                gv[bf], out_h.at[pl.ds(base + c * CH, CH)], so[bf])

        def compute(bf, bi):
            def grp(g, carry):
                r0 = g * 16
                si = ivs[bi][pl.ds(r0, 16)]
                di = ivd[bi][pl.ds(r0, 16)]
                rx = plsc.load_gather(px_v, [si]) - plsc.load_gather(px_v, [di])
                ry = plsc.load_gather(py_v, [si]) - plsc.load_gather(py_v, [di])
                rz = plsc.load_gather(pz_v, [si]) - plsc.load_gather(pz_v, [di])
                d2g = rx * rx + ry * ry + rz * rz
                for i in range(16):
                    r = r0 + i
                    d2s = d2g[i]
                    for j in range(H // 16):
                        sl = pl.ds(j * 16, 16)
                        gv[bf][r, sl] = (av[bf][r, sl] + bv[bf][r, sl]
                                         + d2s * wrow[j])
                return carry

            lax.fori_loop(0, CH // 16, grp, 0)

        def step(c, bf, bi):
            bf1, bi1, bi2 = 1 - bf, (bi + 1) % 4, (bi + 2) % 4
            gat_a(bf, bi).wait()
            gat_b(bf, bi).wait()

            def ahead1():
                ld_is(c + 1, bi1).wait()
                ld_id(c + 1, bi1).wait()
                gat_a(bf1, bi1).start()
                gat_b(bf1, bi1).start()

            def ahead2():
                ld_is(c + 2, bi2).start()
                ld_id(c + 2, bi2).start()

            if isinstance(c, int):
                if c + 1 < NCH:
                    ahead1()
                if c + 2 < NCH:
                    ahead2()
                if c >= 2:
                    owr(c - 2, bf).wait()
            else:
                pl.when(c + 1 < NCH)(ahead1)
                pl.when(c + 2 < NCH)(ahead2)
                owr(c - 2, bf).wait()

            compute(bf, bi)
            owr(c, bf).start()

        ld_is(0, 0).start()
        ld_id(0, 0).start()
        ld_is(1, 1).start()
        ld_id(1, 1).start()
        ld_is(0, 0).wait()
        ld_id(0, 0).wait()
        gat_a(0, 0).start()
        gat_b(0, 0).start()
        for c in range(GHEAD):
            step(c, c % 2, c % 4)

        def body(q, carry):
            c0 = GHEAD + 4 * q
            for r in range(4):
                cr = GHEAD + r
                step(c0 + r, cr % 2, cr % 4)
            return carry

        lax.fori_loop(0, (NCH - GHEAD) // 4, body, 0)
        owr(NCH - 2, (NCH - 2) % 2).wait()
        owr(NCH - 1, (NCH - 1) % 2).wait()

    return k(a, b, src, dst, px, py, pz, wd2)


def _scatter_pass(m2, dst, zeros_n):
    @functools.partial(
        pl.kernel,
        out_type=jax.ShapeDtypeStruct((NC, NPAD, H), F32),
        mesh=_sc_mesh(),
        scratch_types=[
            [pltpu.VMEM((CH, H), F32)] * NB,
            [pltpu.VMEM((CH,), I32)] * NB,
            pltpu.VMEM_SHARED((NPAD, H), F32),
            [pltpu.SemaphoreType.DMA] * NB,
            [pltpu.SemaphoreType.DMA] * NB,
            [pltpu.SemaphoreType.DMA] * NB,
        ],
    )
    def k(m2_h, dst_h, zz_h, out_h, mv, iv, agg_sh, sm, si, ss):
        cid = lax.axis_index("c")
        sid = lax.axis_index("s")
        base = (cid * NS + sid) * EPW
        pltpu.sync_copy(zz_h, agg_sh.at[pl.ds(sid * NPS, NPS)])
        plsc.subcore_barrier()

        def ld_m(c, bf):
            return pltpu.make_async_copy(
                m2_h.at[pl.ds(base + c * CH, CH)], mv[bf], sm[bf])

        def ld_i(c, bf):
            return pltpu.make_async_copy(
                dst_h.at[pl.ds(base + c * CH, CH)], iv[bf], si[bf])

        def scat(bf):
            return pltpu.make_async_copy(mv[bf], agg_sh.at[iv[bf]], ss[bf])

        def issue(c, bf):
            ld_m(c, bf).start()
            ld_i(c, bf).start()

        def step(c, bf):
            ld_m(c, bf).wait()
            ld_i(c, bf).wait()
            nb = (bf + NB - 1) % NB

            if isinstance(c, int):
                if c + NB - 1 < NCH:
                    if c >= 1:
                        scat(nb).wait()
                    issue(c + NB - 1, nb)
            else:
                @pl.when(c + NB - 1 < NCH)
                def _():
                    scat(nb).wait()
                    issue(c + NB - 1, nb)

            scat(bf).start(add=True)

        for c in range(NB - 1):
            issue(c, c)
        for c in range(NB - 1):
            step(c, c)

        def body(q, carry):
            c0 = (NB - 1) + NB * q
            for r in range(NB):
                step(c0 + r, (NB - 1 + r) % NB)
            return carry

        lax.fori_loop(0, (NCH - (NB - 1)) // NB, body, 0)
        for bf in range(NB):
            scat(bf).wait()
        plsc.subcore_barrier()
        pltpu.sync_copy(agg_sh.at[pl.ds(sid * NPS, NPS)],
                        out_h.at[cid, pl.ds(sid * NPS, NPS)])

    return k(m2, dst, zeros_n)


BRN = 2000
BRE = 2560


DE = 2 * H + 1 + ED


def _tc_prep0(x, we1):
    def body(x_ref, w_ref, a_ref, b_ref):
        xv = x_ref[...]
        w = w_ref[0]
        a_ref[...] = jnp.dot(xv, w[:H, :], preferred_element_type=F32)
        b_ref[...] = jnp.dot(xv, w[H:2 * H, :], preferred_element_type=F32)

    return pl.pallas_call(
        body,
        grid=(N // BRN,),
        in_specs=[
            pl.BlockSpec((BRN, H), lambda i: (i, 0)),
            pl.BlockSpec((1, DE, H), lambda i: (0, 0, 0)),
        ],
        out_specs=[pl.BlockSpec((BRN, H), lambda i: (i, 0))] * 2,
        out_shape=[jax.ShapeDtypeStruct((N, H), F32)] * 2,
    )(x, we1)


def _tc_edge(l, g, eat, wpack, eprm):
    def body(g_ref, ea_ref, w_ref, prm_ref, o_ref):
        w = w_ref[0]
        prm = prm_ref[0]
        cont = lax.dot_general(ea_ref[...], w[:ED, :],
                               (((0,), (0,)), ((), ())),
                               preferred_element_type=F32)
        m1 = g_ref[...] + cont + prm[0:1, :]
        m1 = m1 * lax.logistic(m1)
        m2 = jnp.dot(m1, w[ED:, :], preferred_element_type=F32) + prm[1:2, :]
        o_ref[...] = m2 * lax.logistic(m2)

    return pl.pallas_call(
        body,
        grid=(E // BRE,),
        in_specs=[
            pl.BlockSpec((BRE, H), lambda i: (i, 0)),
            pl.BlockSpec((ED, BRE), lambda i: (0, i)),
            pl.BlockSpec((1, ED + H, H), lambda i, _l=l: (_l, 0, 0)),
            pl.BlockSpec((1, 2, H), lambda i, _l=l: (_l, 0, 0)),
        ],
        out_specs=pl.BlockSpec((BRE, H), lambda i: (i, 0)),
        out_shape=jax.ShapeDtypeStruct((E, H), F32),
    )(g, eat, wpack, eprm)


def _tc_node(l, h, aggp, wn1, wn2, nprm, we1=None):
    with_prep = we1 is not None

    def body(h_ref, ag_ref, wn1_ref, wn2_ref, prm_ref, *rest):
        wn1 = wn1_ref[0]
        prm = prm_ref[0]
        agg = ag_ref[0] + ag_ref[1]
        t = (jnp.dot(h_ref[...], wn1[:H, :], preferred_element_type=F32)
             + jnp.dot(agg, wn1[H:, :], preferred_element_type=F32)
             + prm[0:1, :])
        t = t * lax.logistic(t)
        hn = jnp.dot(t, wn2_ref[0], preferred_element_type=F32) + prm[1:2, :]
        if with_prep:
            we1_ref, hn_ref, a_ref, b_ref = rest
            w = we1_ref[0]
            hn_ref[...] = hn
            a_ref[...] = jnp.dot(hn, w[:H, :], preferred_element_type=F32)
            b_ref[...] = jnp.dot(hn, w[H:2 * H, :], preferred_element_type=F32)
        else:
            (hn_ref,) = rest
            hn_ref[...] = hn

    blk = lambda i: (i, 0)
    lsel = lambda shape: pl.BlockSpec(shape, lambda i, _l=l: (_l, 0, 0))
    in_specs = [
        pl.BlockSpec((BRN, H), blk),
        pl.BlockSpec((NC, BRN, H), lambda i: (0, i, 0)),
        lsel((1, 2 * H, H)),
        lsel((1, H, H)),
        lsel((1, 2, H)),
    ]
    args = [h, aggp, wn1, wn2, nprm]
    if with_prep:
        in_specs.append(pl.BlockSpec((1, DE, H), lambda i, _l=l: (_l + 1, 0, 0)))
        args.append(we1)
        out_specs = [pl.BlockSpec((BRN, H), blk)] * 3
        out_shape = [jax.ShapeDtypeStruct((N, H), F32)] * 3
    else:
        out_specs = pl.BlockSpec((BRN, H), blk)
        out_shape = jax.ShapeDtypeStruct((N, H), F32)

    return pl.pallas_call(
        body,
        grid=(N // BRN,),
        in_specs=in_specs,
        out_specs=out_specs,
        out_shape=out_shape,
    )(*args)


def kernel(x, pos, edge_index, edge_attr, We1, be1, We2, be2, Wn1, bn1, Wn2, bn2):
    src = edge_index[0]
    dst = edge_index[1]
    px = pos[:, 0]
    py = pos[:, 1]
    pz = pos[:, 2]
    zeros_t = jnp.zeros((NPS, H), F32)
    eat = edge_attr.T
    wpack = jnp.concatenate([We1[:, 2 * H + 1:], We2], axis=1)
    eprm = jnp.stack([be1, be2], axis=1)
    nprm = jnp.stack([bn1, bn2], axis=1)
    wd2 = We1[:, 2 * H]

    a, b = _tc_prep0(x, We1)
    h = x
    for l in range(L):
        g = _gather_pass(a, b, src, dst, px, py, pz, wd2[l])
        m2 = _tc_edge(l, g, eat, wpack, eprm)
        aggp = _scatter_pass(m2, dst, zeros_t)
        if l < L - 1:
            h, a, b = _tc_node(l, h, aggp, Wn1, Wn2, nprm, We1)
        else:
            h = _tc_node(l, h, aggp, Wn1, Wn2, nprm)
    return h, pos


# speedup vs baseline: 1.2518x; 1.1185x over previous
import functools

import jax
import jax.numpy as jnp
from jax import lax
from jax.experimental import pallas as pl
from jax.experimental.pallas import tpu as pltpu
from jax.experimental.pallas import tpu_sc as plsc

N = 10000
E = 320000
H = 128
ED = 16
L = 2

NC = 2
NS = 16
NW = NC * NS
EPW = E // NW
CH = 80
NCH = EPW // CH
NPAD = 10240
NPS = NPAD // NS

F32 = jnp.float32
I32 = jnp.int32


def _sc_mesh():
    return plsc.VectorSubcoreMesh(
        core_axis_name="c", subcore_axis_name="s",
        num_cores=NC, num_subcores=NS)


NB = 3

E0S = (0, 158720)
EPWS = (158720 // NW, 161280 // NW)


def _gather_pass(a, b, src, dst, px, py, pz, wd2, e0, epw):
    he = epw * NW
    nch = epw // CH
    head = 2 + ((nch - 2) % 4)

    @functools.partial(
        pl.kernel,
        out_type=jax.ShapeDtypeStruct((he, H), F32),
        mesh=_sc_mesh(),
        scratch_types=[
            pltpu.VMEM((N,), F32),
            pltpu.VMEM((N,), F32),
            pltpu.VMEM((N,), F32),
            pltpu.VMEM((H,), F32),
            [pltpu.VMEM((CH, H), F32)] * 2,
            [pltpu.VMEM((CH, H), F32)] * 2,
            [pltpu.VMEM((CH, H), F32)] * 2,
            [pltpu.VMEM((CH,), I32)] * 4,
            [pltpu.VMEM((CH,), I32)] * 4,
            [pltpu.SemaphoreType.DMA] * 2,
            [pltpu.SemaphoreType.DMA] * 2,
            [pltpu.SemaphoreType.DMA] * 2,
            [pltpu.SemaphoreType.DMA] * 4,
            [pltpu.SemaphoreType.DMA] * 4,
        ],
        compiler_params=pltpu.CompilerParams(needs_layout_passes=False),
    )
    def k(a_h, b_h, src_h, dst_h, px_h, py_h, pz_h, w_h, out_h,
          px_v, py_v, pz_v, w_v, av, bv, gv, ivs, ivd,
          sga, sgb, so, sis, sid_):
        cid = lax.axis_index("c")
        sid = lax.axis_index("s")
        base = (cid * NS + sid) * epw
        bg = e0 + base
        pltpu.sync_copy(px_h, px_v)
        pltpu.sync_copy(py_h, py_v)
        pltpu.sync_copy(pz_h, pz_v)
        pltpu.sync_copy(w_h, w_v)
        wrow = [w_v[pl.ds(j * 16, 16)] for j in range(H // 16)]

        def ld_is(c, bi):
            return pltpu.make_async_copy(
                src_h.at[pl.ds(bg + c * CH, CH)], ivs[bi], sis[bi])

        def ld_id(c, bi):
            return pltpu.make_async_copy(
                dst_h.at[pl.ds(bg + c * CH, CH)], ivd[bi], sid_[bi])

        def gat_a(bf, bi):
            return pltpu.make_async_copy(a_h.at[ivs[bi]], av[bf], sga[bf])

        def gat_b(bf, bi):
            return pltpu.make_async_copy(b_h.at[ivd[bi]], bv[bf], sgb[bf])

        def owr(c, bf):
            return pltpu.make_async_copy(
                gv[bf], out_h.at[pl.ds(base + c * CH, CH)], so[bf])

        def compute(bf, bi):
            def grp(g, carry):
                r0 = g * 16
                si = ivs[bi][pl.ds(r0, 16)]
                di = ivd[bi][pl.ds(r0, 16)]
                rx = plsc.load_gather(px_v, [si]) - plsc.load_gather(px_v, [di])
                ry = plsc.load_gather(py_v, [si]) - plsc.load_gather(py_v, [di])
                rz = plsc.load_gather(pz_v, [si]) - plsc.load_gather(pz_v, [di])
                d2g = rx * rx + ry * ry + rz * rz
                for i in range(16):
                    r = r0 + i
                    d2s = d2g[i]
                    for j in range(H // 16):
                        sl = pl.ds(j * 16, 16)
                        gv[bf][r, sl] = (av[bf][r, sl] + bv[bf][r, sl]
                                         + d2s * wrow[j])
                return carry

            lax.fori_loop(0, CH // 16, grp, 0)

        def step(c, bf, bi):
            bf1, bi1, bi2 = 1 - bf, (bi + 1) % 4, (bi + 2) % 4
            gat_a(bf, bi).wait()
            gat_b(bf, bi).wait()

            def ahead1():
                ld_is(c + 1, bi1).wait()
                ld_id(c + 1, bi1).wait()
                gat_a(bf1, bi1).start()
                gat_b(bf1, bi1).start()

            def ahead2():
                ld_is(c + 2, bi2).start()
                ld_id(c + 2, bi2).start()

            if isinstance(c, int):
                if c + 1 < nch:
                    ahead1()
                if c + 2 < nch:
                    ahead2()
                if c >= 2:
                    owr(c - 2, bf).wait()
            else:
                pl.when(c + 1 < nch)(ahead1)
                pl.when(c + 2 < nch)(ahead2)
                owr(c - 2, bf).wait()

            compute(bf, bi)
            owr(c, bf).start()

        ld_is(0, 0).start()
        ld_id(0, 0).start()
        ld_is(1, 1).start()
        ld_id(1, 1).start()
        ld_is(0, 0).wait()
        ld_id(0, 0).wait()
        gat_a(0, 0).start()
        gat_b(0, 0).start()
        for c in range(head):
            step(c, c % 2, c % 4)

        def body(q, carry):
            c0 = head + 4 * q
            for r in range(4):
                cr = head + r
                step(c0 + r, cr % 2, cr % 4)
            return carry

        lax.fori_loop(0, (nch - head) // 4, body, 0)
        owr(nch - 2, (nch - 2) % 2).wait()
        owr(nch - 1, (nch - 1) % 2).wait()

    return k(a, b, src, dst, px, py, pz, wd2)


def _scatter_pass(m2, dst, zeros_n, e0, epw):
    nch = epw // CH
    prime = 2 + ((nch - 2) % NB)

    @functools.partial(
        pl.kernel,
        out_type=jax.ShapeDtypeStruct((NC, NPAD, H), F32),
        mesh=_sc_mesh(),
        scratch_types=[
            [pltpu.VMEM((CH, H), F32)] * NB,
            [pltpu.VMEM((CH,), I32)] * NB,
            pltpu.VMEM_SHARED((NPAD, H), F32),
            [pltpu.SemaphoreType.DMA] * NB,
            [pltpu.SemaphoreType.DMA] * NB,
            [pltpu.SemaphoreType.DMA] * NB,
        ],
    )
    def k(m2_h, dst_h, zz_h, out_h, mv, iv, agg_sh, sm, si, ss):
        cid = lax.axis_index("c")
        sid = lax.axis_index("s")
        base = (cid * NS + sid) * epw
        bg = e0 + base
        pltpu.sync_copy(zz_h, agg_sh.at[pl.ds(sid * NPS, NPS)])
        plsc.subcore_barrier()

        def ld_m(c, bf):
            return pltpu.make_async_copy(
                m2_h.at[pl.ds(base + c * CH, CH)], mv[bf], sm[bf])

        def ld_i(c, bf):
            return pltpu.make_async_copy(
                dst_h.at[pl.ds(bg + c * CH, CH)], iv[bf], si[bf])

        def scat(bf):
            return pltpu.make_async_copy(mv[bf], agg_sh.at[iv[bf]], ss[bf])

        def issue(c, bf):
            ld_m(c, bf).start()
            ld_i(c, bf).start()

        def step(c, bf):
            ld_m(c, bf).wait()
            ld_i(c, bf).wait()
            nb = (bf + NB - 1) % NB

            if isinstance(c, int):
                if c + NB - 1 < nch:
                    if c >= 1:
                        scat(nb).wait()
                    issue(c + NB - 1, nb)
            else:
                @pl.when(c + NB - 1 < nch)
                def _():
                    scat(nb).wait()
                    issue(c + NB - 1, nb)

            scat(bf).start(add=True)

        for c in range(min(NB - 1, prime)):
            issue(c, c)
        for c in range(prime):
            step(c, c % NB)

        def body(q, carry):
            c0 = prime + NB * q
            for r in range(NB):
                step(c0 + r, (prime + r) % NB)
            return carry

        lax.fori_loop(0, (nch - prime) // NB, body, 0)
        for bf in range(NB):
            scat(bf).wait()
        plsc.subcore_barrier()
        pltpu.sync_copy(agg_sh.at[pl.ds(sid * NPS, NPS)],
                        out_h.at[cid, pl.ds(sid * NPS, NPS)])

    return k(m2, dst, zeros_n)


BRN = 2000
BRE = 2560


DE = 2 * H + 1 + ED


def _tc_prep0(x, we1):
    def body(x_ref, w_ref, a_ref, b_ref):
        xv = x_ref[...]
        w = w_ref[0]
        a_ref[...] = jnp.dot(xv, w[:H, :], preferred_element_type=F32)
        b_ref[...] = jnp.dot(xv, w[H:2 * H, :], preferred_element_type=F32)

    return pl.pallas_call(
        body,
        grid=(N // BRN,),
        in_specs=[
            pl.BlockSpec((BRN, H), lambda i: (i, 0)),
            pl.BlockSpec((1, DE, H), lambda i: (0, 0, 0)),
        ],
        out_specs=[pl.BlockSpec((BRN, H), lambda i: (i, 0))] * 2,
        out_shape=[jax.ShapeDtypeStruct((N, H), F32)] * 2,
    )(x, we1)


def _tc_edge(l, g, eat, wpack, eprm, e0):
    def body(g_ref, ea_ref, w_ref, prm_ref, o_ref):
        w = w_ref[0]
        prm = prm_ref[0]
        cont = lax.dot_general(ea_ref[...], w[:ED, :],
                               (((0,), (0,)), ((), ())),
                               preferred_element_type=F32)
        m1 = g_ref[...] + cont + prm[0:1, :]
        m1 = m1 * lax.logistic(m1)
        m2 = jnp.dot(m1, w[ED:, :], preferred_element_type=F32) + prm[1:2, :]
        o_ref[...] = m2 * lax.logistic(m2)

    he = g.shape[0]
    col0 = e0 // BRE

    return pl.pallas_call(
        body,
        grid=(he // BRE,),
        in_specs=[
            pl.BlockSpec((BRE, H), lambda i: (i, 0)),
            pl.BlockSpec((ED, BRE), lambda i, _c=col0: (0, i + _c)),
            pl.BlockSpec((1, ED + H, H), lambda i, _l=l: (_l, 0, 0)),
            pl.BlockSpec((1, 2, H), lambda i, _l=l: (_l, 0, 0)),
        ],
        out_specs=pl.BlockSpec((BRE, H), lambda i: (i, 0)),
        out_shape=jax.ShapeDtypeStruct((he, H), F32),
    )(g, eat, wpack, eprm)


def _tc_node(l, h, aggp0, aggp1, wn1, wn2, nprm, we1=None):
    with_prep = we1 is not None

    def body(h_ref, ag0_ref, ag1_ref, wn1_ref, wn2_ref, prm_ref, *rest):
        wn1 = wn1_ref[0]
        prm = prm_ref[0]
        agg = (ag0_ref[0] + ag0_ref[1]) + (ag1_ref[0] + ag1_ref[1])
        t = (jnp.dot(h_ref[...], wn1[:H, :], preferred_element_type=F32)
             + jnp.dot(agg, wn1[H:, :], preferred_element_type=F32)
             + prm[0:1, :])
        t = t * lax.logistic(t)
        hn = jnp.dot(t, wn2_ref[0], preferred_element_type=F32) + prm[1:2, :]
        if with_prep:
            we1_ref, hn_ref, a_ref, b_ref = rest
            w = we1_ref[0]
            hn_ref[...] = hn
            a_ref[...] = jnp.dot(hn, w[:H, :], preferred_element_type=F32)
            b_ref[...] = jnp.dot(hn, w[H:2 * H, :], preferred_element_type=F32)
        else:
            (hn_ref,) = rest
            hn_ref[...] = hn

    blk = lambda i: (i, 0)
    lsel = lambda shape: pl.BlockSpec(shape, lambda i, _l=l: (_l, 0, 0))
    in_specs = [
        pl.BlockSpec((BRN, H), blk),
        pl.BlockSpec((NC, BRN, H), lambda i: (0, i, 0)),
        pl.BlockSpec((NC, BRN, H), lambda i: (0, i, 0)),
        lsel((1, 2 * H, H)),
        lsel((1, H, H)),
        lsel((1, 2, H)),
    ]
    args = [h, aggp0, aggp1, wn1, wn2, nprm]
    if with_prep:
        in_specs.append(pl.BlockSpec((1, DE, H), lambda i, _l=l: (_l + 1, 0, 0)))
        args.append(we1)
        out_specs = [pl.BlockSpec((BRN, H), blk)] * 3
        out_shape = [jax.ShapeDtypeStruct((N, H), F32)] * 3
    else:
        out_specs = pl.BlockSpec((BRN, H), blk)
        out_shape = jax.ShapeDtypeStruct((N, H), F32)

    return pl.pallas_call(
        body,
        grid=(N // BRN,),
        in_specs=in_specs,
        out_specs=out_specs,
        out_shape=out_shape,
    )(*args)


def kernel(x, pos, edge_index, edge_attr, We1, be1, We2, be2, Wn1, bn1, Wn2, bn2):
    src = edge_index[0]
    dst = edge_index[1]
    px = pos[:, 0]
    py = pos[:, 1]
    pz = pos[:, 2]
    zeros_t = jnp.zeros((NPS, H), F32)
    eat = edge_attr.T
    wpack = jnp.concatenate([We1[:, 2 * H + 1:], We2], axis=1)
    eprm = jnp.stack([be1, be2], axis=1)
    nprm = jnp.stack([bn1, bn2], axis=1)
    wd2 = We1[:, 2 * H]

    a, b = _tc_prep0(x, We1)
    h = x
    for l in range(L):
        g0 = _gather_pass(a, b, src, dst, px, py, pz, wd2[l], E0S[0], EPWS[0])
        g1 = _gather_pass(a, b, src, dst, px, py, pz, wd2[l], E0S[1], EPWS[1])
        m20 = _tc_edge(l, g0, eat, wpack, eprm, E0S[0])
        m21 = _tc_edge(l, g1, eat, wpack, eprm, E0S[1])
        ag0 = _scatter_pass(m20, dst, zeros_t, E0S[0], EPWS[0])
        ag1 = _scatter_pass(m21, dst, zeros_t, E0S[1], EPWS[1])
        if l < L - 1:
            h, a, b = _tc_node(l, h, ag0, ag1, Wn1, Wn2, nprm, We1)
        else:
            h = _tc_node(l, h, ag0, ag1, Wn1, Wn2, nprm)
    return h, pos
